# baseline (speedup 1.0000x reference)
import functools

import jax
import jax.numpy as jnp
from jax import lax
from jax.experimental import pallas as pl
from jax.experimental.pallas import tpu as pltpu
from jax.experimental.pallas import tpu_sc as plsc

_K = 5
_B = 128
_N = 32768
_L = 16
_GV = 32
_NB = _N // (_L * _GV)
_U = 8
_RPW = _B // 32


def _insert_v(ms, v):
    out = []
    for m in ms:
        out.append(jnp.maximum(m, v))
        v = jnp.minimum(m, v)
    return tuple(out)


def _insert_vi(ms, ids, v, iv):
    out_m, out_i = [], []
    for m, im in zip(ms, ids):
        c = v > m
        out_m.append(jnp.where(c, v, m))
        out_i.append(jnp.where(c, iv, im))
        v, iv = jnp.where(c, m, v), jnp.where(c, im, iv)
    return tuple(out_m), tuple(out_i)


def _row_top5_idxvec(rbuf, gbuf, lane, dmas, fire_next):
    neg = jnp.full((_L,), -jnp.inf, jnp.float32)
    zero_i = jnp.zeros((_L,), jnp.int32)

    def blockmax(blk, bms):
        base = blk * (_GV * _L)
        bm = rbuf[pl.ds(base, _L)]
        for u in range(1, _GV):
            bm = jnp.maximum(bm, rbuf[pl.ds(base + u * _L, _L)])
        gbuf[pl.ds(blk * _L, _L)] = bm
        return _insert_v(bms, bm)

    dmas[0].wait()
    bms = lax.fori_loop(0, _NB // 2, blockmax, (neg,) * _K)
    dmas[1].wait()
    if fire_next is not None:
        fire_next()
    bms = lax.fori_loop(_NB // 2, _NB, blockmax, bms)

    tau = jnp.float32(jnp.inf)
    for _ in range(_K):
        w = neg
        for m in bms:
            w = jnp.maximum(w, jnp.where(m < tau, m, -jnp.inf))
        tau = jnp.max(w)

    def scan_quad(q, carry):
        gms = [gbuf[pl.ds((q * 4 + b) * _L, _L)] for b in range(4)]
        qm = jnp.maximum(jnp.maximum(gms[0], gms[1]),
                         jnp.maximum(gms[2], gms[3]))
        qtrig = jnp.any(qm >= tau)

        def qslow(qargs):
            def per_blk(blk, args):
                gm = gbuf[pl.ds(blk * _L, _L)]
                trig = jnp.any(gm >= tau)

                def slow(args):
                    base = blk * (_GV * _L)
                    subs = []
                    for s in range(_GV // _U):
                        sm = rbuf[pl.ds(base + s * _U * _L, _L)]
                        for u in range(1, _U):
                            sm = jnp.maximum(
                                sm, rbuf[pl.ds(base + (s * _U + u) * _L, _L)])
                        subs.append(sm)
                    for s in range(_GV // _U):
                        strig = jnp.any(subs[s] >= tau)

                        def sslow(a, s=s):
                            def chunk(ci, a):
                                ms, ids = a[:_K], a[_K:]
                                cb = base + (s * _U + ci * (_U // 2)) * _L
                                for u in range(_U // 2):
                                    v = rbuf[pl.ds(cb + u * _L, _L)]
                                    iv = lane + (cb + u * _L)
                                    ms, ids = _insert_vi(ms, ids, v, iv)
                                return (*ms, *ids)

                            return lax.fori_loop(0, 2, chunk, a)

                        args = lax.cond(strig, sslow, lambda a: a, args)
                    return args

                return lax.cond(trig, slow, lambda a: a, args)

            return lax.fori_loop(q * 4, q * 4 + 4, per_blk, qargs)

        return lax.cond(qtrig, qslow, lambda a: a, carry)

    carry = lax.fori_loop(
        0, _NB // 4, scan_quad, ((neg,) * _K) + ((zero_i,) * _K))
    ms, ids = list(carry[:_K]), list(carry[_K:])

    big = jnp.int32(1 << 30)
    idxvec = zero_i
    for k in range(_K):
        w = ms[0]
        for m in ms[1:]:
            w = jnp.maximum(w, m)
        mval = jnp.max(w)
        wi = jnp.where(ms[0] == mval, ids[0], big)
        for m, im in zip(ms[1:], ids[1:]):
            wi = jnp.minimum(wi, jnp.where(m == mval, im, big))
        imin = jnp.min(wi)
        for j in range(_K):
            ms[j] = jnp.where(ids[j] == imin, -jnp.inf, ms[j])
        idxvec = jnp.where(lane == k, imin, idxvec)
    return idxvec


def _wta_body(x_hbm, out_hbm, buf0, buf1, zbuf, gbuf, sem_in, sem_in2,
              sem_out):
    nc = 2
    wid = lax.axis_index("s") * nc + lax.axis_index("c")
    row0 = wid * _RPW
    bufs = (buf0, buf1)

    lane = lax.iota(jnp.int32, _L)
    ones_v = jnp.full((_L,), 1.0, jnp.float32)
    zeros_v = jnp.zeros((_L,), jnp.float32)
    mask5 = lane < _K

    half = _N // 2

    def row_copy(row, buf):
        return (
            pltpu.async_copy(
                x_hbm.at[row, pl.ds(0, half)], buf.at[pl.ds(0, half)],
                sem_in),
            pltpu.async_copy(
                x_hbm.at[row, pl.ds(half, half)], buf.at[pl.ds(half, half)],
                sem_in2),
        )

    cur_dma = row_copy(row0, buf0)

    def zinit(i, c):
        for u in range(_U):
            zbuf[pl.ds((i * _U + u) * _L, _L)] = zeros_v
        return c

    lax.fori_loop(0, _N // (_L * _U), zinit, 0)

    out_dma = None
    prev_idxvec = None
    for j in range(_RPW):
        fired = {}
        if j + 1 < _RPW:
            def fire(fired=fired, row=row0 + j + 1, b=bufs[(j + 1) % 2]):
                fired["d"] = row_copy(row, b)
        else:
            fire = None
        idxvec = _row_top5_idxvec(bufs[j % 2], gbuf, lane, cur_dma, fire)
        if fire is not None:
            cur_dma = fired["d"]
        if out_dma is not None:
            out_dma.wait()
            plsc.store_scatter(zbuf, [prev_idxvec], zeros_v, mask=mask5)
        plsc.store_scatter(zbuf, [idxvec], ones_v, mask=mask5)
        out_dma = pltpu.async_copy(zbuf, out_hbm.at[row0 + j], sem_out)
        prev_idxvec = idxvec
    out_dma.wait()


def kernel(x):
    mesh = plsc.VectorSubcoreMesh(core_axis_name="c", subcore_axis_name="s")
    run = functools.partial(
        pl.kernel,
        mesh=mesh,
        out_type=jax.ShapeDtypeStruct((_B, _N), jnp.float32),
        scratch_types=[
            pltpu.VMEM((_N,), jnp.float32),
            pltpu.VMEM((_N,), jnp.float32),
            pltpu.VMEM((_N,), jnp.float32),
            pltpu.VMEM((_NB * _L,), jnp.float32),
            pltpu.SemaphoreType.DMA,
            pltpu.SemaphoreType.DMA,
            pltpu.SemaphoreType.DMA,
        ],
        compiler_params=pltpu.CompilerParams(needs_layout_passes=False),
    )(_wta_body)
    return run(x)


# speedup vs baseline: 1.0439x; 1.0439x over previous
import functools

import jax
import jax.numpy as jnp
from jax import lax
from jax.experimental import pallas as pl
from jax.experimental.pallas import tpu as pltpu
from jax.experimental.pallas import tpu_sc as plsc

_K = 5
_B = 128
_N = 32768
_L = 16
_GV = 32
_NB = _N // (_L * _GV)
_U = 8
_RPW = _B // 32


def _insert_v(ms, v):
    out = []
    for m in ms:
        out.append(jnp.maximum(m, v))
        v = jnp.minimum(m, v)
    return tuple(out)


def _insert_vi(ms, ids, v, iv):
    out_m, out_i = [], []
    for m, im in zip(ms, ids):
        c = v > m
        out_m.append(jnp.where(c, v, m))
        out_i.append(jnp.where(c, iv, im))
        v, iv = jnp.where(c, m, v), jnp.where(c, im, iv)
    return tuple(out_m), tuple(out_i)


def _row_top5_idxvec(rbuf, gbuf, lane):
    neg = jnp.full((_L,), -jnp.inf, jnp.float32)
    zero_i = jnp.zeros((_L,), jnp.int32)

    def blockmax(blk, bms):
        base = blk * (_GV * _L)
        bm = rbuf[pl.ds(base, _L)]
        for u in range(1, _GV):
            bm = jnp.maximum(bm, rbuf[pl.ds(base + u * _L, _L)])
        gbuf[pl.ds(blk * _L, _L)] = bm
        return _insert_v(bms, bm)

    bms = lax.fori_loop(0, _NB, blockmax, (neg,) * _K)

    tau = jnp.float32(jnp.inf)
    for _ in range(_K):
        w = neg
        for m in bms:
            w = jnp.maximum(w, jnp.where(m < tau, m, -jnp.inf))
        tau = jnp.max(w)

    def scan_quad(q, carry):
        gms = [gbuf[pl.ds((q * 4 + b) * _L, _L)] for b in range(4)]
        qm = jnp.maximum(jnp.maximum(gms[0], gms[1]),
                         jnp.maximum(gms[2], gms[3]))
        qtrig = jnp.any(qm >= tau)

        def qslow(qargs):
            def per_blk(blk, args):
                gm = gbuf[pl.ds(blk * _L, _L)]
                trig = jnp.any(gm >= tau)

                def slow(args):
                    base = blk * (_GV * _L)
                    subs = []
                    for s in range(_GV // _U):
                        sm = rbuf[pl.ds(base + s * _U * _L, _L)]
                        for u in range(1, _U):
                            sm = jnp.maximum(
                                sm, rbuf[pl.ds(base + (s * _U + u) * _L, _L)])
                        subs.append(sm)
                    for s in range(_GV // _U):
                        strig = jnp.any(subs[s] >= tau)

                        def sslow(a, s=s):
                            def chunk(ci, a):
                                ms, ids = a[:_K], a[_K:]
                                cb = base + (s * _U + ci * (_U // 2)) * _L
                                for u in range(_U // 2):
                                    v = rbuf[pl.ds(cb + u * _L, _L)]
                                    iv = lane + (cb + u * _L)
                                    ms, ids = _insert_vi(ms, ids, v, iv)
                                return (*ms, *ids)

                            return lax.fori_loop(0, 2, chunk, a)

                        args = lax.cond(strig, sslow, lambda a: a, args)
                    return args

                return lax.cond(trig, slow, lambda a: a, args)

            return lax.fori_loop(q * 4, q * 4 + 4, per_blk, qargs)

        return lax.cond(qtrig, qslow, lambda a: a, carry)

    carry = lax.fori_loop(
        0, _NB // 4, scan_quad, ((neg,) * _K) + ((zero_i,) * _K))
    ms, ids = list(carry[:_K]), list(carry[_K:])

    big = jnp.int32(1 << 30)
    idxvec = zero_i
    for k in range(_K):
        w = ms[0]
        for m in ms[1:]:
            w = jnp.maximum(w, m)
        mval = jnp.max(w)
        wi = jnp.where(ms[0] == mval, ids[0], big)
        for m, im in zip(ms[1:], ids[1:]):
            wi = jnp.minimum(wi, jnp.where(m == mval, im, big))
        imin = jnp.min(wi)
        for j in range(_K):
            ms[j] = jnp.where(ids[j] == imin, -jnp.inf, ms[j])
        idxvec = jnp.where(lane == k, imin, idxvec)
    return idxvec


def _wta_body(x_hbm, out_hbm, buf0, buf1, zbuf, gbuf, sem_in, sem_out):
    nc = 2
    wid = lax.axis_index("s") * nc + lax.axis_index("c")
    row0 = wid * _RPW
    bufs = (buf0, buf1)

    lane = lax.iota(jnp.int32, _L)
    ones_v = jnp.full((_L,), 1.0, jnp.float32)
    zeros_v = jnp.zeros((_L,), jnp.float32)
    mask5 = lane < _K

    in_dma = pltpu.async_copy(x_hbm.at[row0], buf0, sem_in)

    def zinit(i, c):
        for u in range(_U):
            zbuf[pl.ds((i * _U + u) * _L, _L)] = zeros_v
        return c

    lax.fori_loop(0, _N // (_L * _U), zinit, 0)

    out_dma = None
    prev_idxvec = None
    for j in range(_RPW):
        in_dma.wait()
        if j + 1 < _RPW:
            in_dma = pltpu.async_copy(
                x_hbm.at[row0 + j + 1], bufs[(j + 1) % 2], sem_in)
        idxvec = _row_top5_idxvec(bufs[j % 2], gbuf, lane)
        if out_dma is not None:
            out_dma.wait()
            plsc.store_scatter(zbuf, [prev_idxvec], zeros_v, mask=mask5)
        plsc.store_scatter(zbuf, [idxvec], ones_v, mask=mask5)
        out_dma = pltpu.async_copy(zbuf, out_hbm.at[row0 + j], sem_out)
        prev_idxvec = idxvec
    out_dma.wait()


def kernel(x):
    mesh = plsc.VectorSubcoreMesh(core_axis_name="c", subcore_axis_name="s")
    run = functools.partial(
        pl.kernel,
        mesh=mesh,
        out_type=jax.ShapeDtypeStruct((_B, _N), jnp.float32),
        scratch_types=[
            pltpu.VMEM((_N,), jnp.float32),
            pltpu.VMEM((_N,), jnp.float32),
            pltpu.VMEM((_N,), jnp.float32),
            pltpu.VMEM((_NB * _L,), jnp.float32),
            pltpu.SemaphoreType.DMA,
            pltpu.SemaphoreType.DMA,
        ],
        compiler_params=pltpu.CompilerParams(needs_layout_passes=False),
    )(_wta_body)
    return run(x)
